# baseline (speedup 1.0000x reference)
import functools

import jax
import jax.numpy as jnp
from jax import lax
from jax.experimental import pallas as pl
from jax.experimental.pallas import tpu as pltpu
from jax.experimental.pallas import tpu_sc as plsc

B = 16384
F = 26
D = 16
N = B * F
V = 2600000
H1 = 400
H2 = 400
DEEP_IN = F * D

SLICES = 4
BS = B // SLICES
NSL = N // SLICES

NC = 2
NS = 16
NW = NC * NS
N_PER_W = NSL // NW
CH = 128
NCHUNK = N_PER_W // CH
NBUF = 2

FC_PER_W = N // NW
FC_NCHUNK = FC_PER_W // CH


def _detile_body(t_ref, out_ref):
    out_ref[...] = jnp.transpose(t_ref[...], (1, 0))


def _tc_detile(emb_t):
    cb = 16384
    grid = ((V + cb - 1) // cb,)
    return pl.pallas_call(
        _detile_body,
        grid=grid,
        in_specs=[pl.BlockSpec((D, cb), lambda i: (0, i))],
        out_specs=pl.BlockSpec((cb, D), lambda i: (i, 0)),
        out_shape=jax.ShapeDtypeStruct((V, D), jnp.float32),
    )(emb_t)


def _sc_emb_body(x_hbm, emb_hbm, emb_out, idx_v, ebuf, esems):
    wid = lax.axis_index("s") * NC + lax.axis_index("c")
    pltpu.sync_copy(x_hbm.at[pl.ds(wid * NCHUNK, NCHUNK)], idx_v)
    base = wid * N_PER_W

    def fire(c, b):
        pltpu.async_copy(emb_hbm.at[idx_v.at[c]], ebuf.at[b], esems[b])

    def drain(c, b):
        pltpu.make_async_copy(emb_hbm.at[idx_v.at[c]], ebuf.at[b], esems[b]).wait()
        pltpu.sync_copy(ebuf.at[b], emb_out.at[pl.ds(base + c * CH, CH)])

    fire(0, 0)

    def step(g, carry):
        for b in range(NBUF):
            c = g * NBUF + b
            nxt = c + 1

            @pl.when(nxt < NCHUNK)
            def _():
                fire(nxt, (b + 1) % NBUF)

            drain(c, b)
        return carry

    lax.fori_loop(0, NCHUNK // NBUF, step, 0, unroll=False)


@functools.cache
def _make_sc_emb():
    return pl.kernel(
        _sc_emb_body,
        out_type=(jax.ShapeDtypeStruct((NSL, D), jnp.float32),),
        mesh=plsc.VectorSubcoreMesh(core_axis_name="c", subcore_axis_name="s",
                                    num_cores=NC, num_subcores=NS),
        compiler_params=pltpu.CompilerParams(use_tc_tiling_on_sc=False),
        scratch_types=[
            pltpu.VMEM((NCHUNK, CH), jnp.int32),
            pltpu.VMEM((NBUF, CH, D), jnp.float32),
            [pltpu.SemaphoreType.DMA] * NBUF,
        ],
    )


def _sc_fc_body(x_hbm, fc_hbm, fc_out, idx_v, fbuf, fsems):
    wid = lax.axis_index("s") * NC + lax.axis_index("c")
    pltpu.sync_copy(x_hbm.at[pl.ds(wid * FC_NCHUNK, FC_NCHUNK)], idx_v)
    base = wid * FC_PER_W

    def fire(c, b):
        pltpu.async_copy(fc_hbm.at[idx_v.at[c]], fbuf.at[b], fsems[b])

    def drain(c, b):
        pltpu.make_async_copy(fc_hbm.at[idx_v.at[c]], fbuf.at[b], fsems[b]).wait()
        pltpu.sync_copy(fbuf.at[b], fc_out.at[pl.ds(base + c * CH, CH)])

    fire(0, 0)

    def step(g, carry):
        for b in range(NBUF):
            c = g * NBUF + b
            nxt = c + 1

            @pl.when(nxt < FC_NCHUNK)
            def _():
                fire(nxt, (b + 1) % NBUF)

            drain(c, b)
        return carry

    lax.fori_loop(0, FC_NCHUNK // NBUF, step, 0, unroll=False)


@functools.cache
def _make_sc_fc():
    return pl.kernel(
        _sc_fc_body,
        out_type=(jax.ShapeDtypeStruct((N,), jnp.float32),),
        mesh=plsc.VectorSubcoreMesh(core_axis_name="c", subcore_axis_name="s",
                                    num_cores=NC, num_subcores=NS),
        compiler_params=pltpu.CompilerParams(use_tc_tiling_on_sc=False),
        scratch_types=[
            pltpu.VMEM((FC_NCHUNK, CH), jnp.int32),
            pltpu.VMEM((NBUF, CH), jnp.float32),
            [pltpu.SemaphoreType.DMA] * NBUF,
        ],
    )


def _tc_body(emb_ref, fc_ref, W1_ref, b1_ref, W2_ref, b2_ref, W3_ref, b3_ref,
             Wlin_ref, blin_ref, out_ref):
    h0 = emb_ref[...]

    j = lax.broadcasted_iota(jnp.int32, (DEEP_IN, D), 0)
    d = lax.broadcasted_iota(jnp.int32, (DEEP_IN, D), 1)
    S = jnp.where(j % D == d, 1.0, 0.0).astype(jnp.float32)
    se = lax.dot_general(h0, S, (((1,), (0,)), ((), ())),
                         preferred_element_type=jnp.float32)
    ss = lax.dot_general(h0 * h0, S, (((1,), (0,)), ((), ())),
                         preferred_element_type=jnp.float32)
    fm = 0.5 * jnp.sum(se * se - ss, axis=1, keepdims=True)

    lin = jnp.sum(fc_ref[...], axis=1, keepdims=True) * Wlin_ref[0, 0] + blin_ref[0, 0]

    h1 = jnp.maximum(
        lax.dot_general(h0, W1_ref[...], (((1,), (1,)), ((), ())),
                        preferred_element_type=jnp.float32) + b1_ref[...], 0.0)
    h2 = jnp.maximum(
        lax.dot_general(h1, W2_ref[...], (((1,), (1,)), ((), ())),
                        preferred_element_type=jnp.float32) + b2_ref[...], 0.0)
    h3 = jnp.sum(h2 * W3_ref[...], axis=1, keepdims=True) + b3_ref[0, 0]

    out_ref[...] = lin + fm + h3


def _tc_forward(emb_flat, fc_mat, W1, b1, W2, b2, W3, b3, W_lin, b_lin):
    bB = 1024
    grid = (BS // bB,)
    return pl.pallas_call(
        _tc_body,
        grid=grid,
        in_specs=[
            pl.BlockSpec((bB, DEEP_IN), lambda i: (i, 0)),
            pl.BlockSpec((bB, F), lambda i: (i, 0)),
            pl.BlockSpec((H1, DEEP_IN), lambda i: (0, 0)),
            pl.BlockSpec((1, H1), lambda i: (0, 0)),
            pl.BlockSpec((H2, H1), lambda i: (0, 0)),
            pl.BlockSpec((1, H2), lambda i: (0, 0)),
            pl.BlockSpec((1, H2), lambda i: (0, 0)),
            pl.BlockSpec(memory_space=pltpu.SMEM),
            pl.BlockSpec(memory_space=pltpu.SMEM),
            pl.BlockSpec(memory_space=pltpu.SMEM),
        ],
        out_specs=pl.BlockSpec((bB, 1), lambda i: (i, 0)),
        out_shape=jax.ShapeDtypeStruct((BS, 1), jnp.float32),
    )(emb_flat, fc_mat, W1, b1, W2, b2, W3, b3, W_lin, b_lin)


def kernel(x, emb_table, fc_table, W_lin, b_lin, W1, b1, W2, b2, W3, b3):
    x_idx = x.astype(jnp.int32).reshape(N // CH, CH)
    fc_flat = fc_table.reshape(-1)
    b1r = b1.reshape(1, H1)
    b2r = b2.reshape(1, H2)
    b3r = b3.reshape(1, 1)
    blinr = b_lin.reshape(1, 1)

    (fc_rows,) = _make_sc_fc()(x_idx, fc_flat)
    emb_rm = _tc_detile(emb_table.T)

    sc = _make_sc_emb()
    rows_per_slice = NSL // CH
    outs = []
    for s in range(SLICES):
        xs = lax.slice_in_dim(x_idx, s * rows_per_slice, (s + 1) * rows_per_slice, axis=0)
        (emb_s,) = sc(xs, emb_rm)
        fc_s = lax.slice_in_dim(fc_rows, s * NSL, (s + 1) * NSL, axis=0)
        outs.append(_tc_forward(
            emb_s.reshape(BS, DEEP_IN), fc_s.reshape(BS, F),
            W1, b1r, W2, b2r, W3, b3r, W_lin, blinr,
        ))
    return jnp.concatenate(outs, axis=0)


# speedup vs baseline: 1.2139x; 1.2139x over previous
import functools

import jax
import jax.numpy as jnp
from jax import lax
from jax.experimental import pallas as pl
from jax.experimental.pallas import tpu as pltpu
from jax.experimental.pallas import tpu_sc as plsc

B = 16384
F = 26
D = 16
N = B * F
H1 = 400
H2 = 400
DEEP_IN = F * D

SLICES = 4
BS = B // SLICES
NSL = N // SLICES

NC = 2
NS = 16
NW = NC * NS
N_PER_W = NSL // NW
CH = 128
NCHUNK = N_PER_W // CH
NBUF = 2


def _sc_gather_body(x_hbm, emb_hbm, fc_hbm, emb_out, fc_out,
                    idx_v, ebuf, fbuf, esems, fsems):
    wid = lax.axis_index("s") * NC + lax.axis_index("c")
    pltpu.sync_copy(x_hbm.at[pl.ds(wid * NCHUNK, NCHUNK)], idx_v)
    base = wid * N_PER_W

    def fire(c, b):
        idx_row = idx_v.at[c]
        pltpu.async_copy(emb_hbm.at[idx_row], ebuf.at[b], esems[b])
        pltpu.async_copy(fc_hbm.at[idx_row], fbuf.at[b], fsems[b])

    def drain(c, b):
        pltpu.make_async_copy(emb_hbm.at[idx_v.at[c]], ebuf.at[b], esems[b]).wait()
        pltpu.make_async_copy(fc_hbm.at[idx_v.at[c]], fbuf.at[b], fsems[b]).wait()
        pltpu.sync_copy(ebuf.at[b], emb_out.at[pl.ds(base + c * CH, CH)])
        pltpu.sync_copy(fbuf.at[b], fc_out.at[pl.ds(base + c * CH, CH)])

    fire(0, 0)

    def step(g, carry):
        for b in range(NBUF):
            c = g * NBUF + b
            nxt = c + 1

            @pl.when(nxt < NCHUNK)
            def _():
                fire(nxt, (b + 1) % NBUF)

            drain(c, b)
        return carry

    lax.fori_loop(0, NCHUNK // NBUF, step, 0, unroll=False)


@functools.cache
def _make_sc_gather():
    return pl.kernel(
        _sc_gather_body,
        out_type=(
            jax.ShapeDtypeStruct((NSL, D), jnp.float32),
            jax.ShapeDtypeStruct((NSL,), jnp.float32),
        ),
        mesh=plsc.VectorSubcoreMesh(core_axis_name="c", subcore_axis_name="s",
                                    num_cores=NC, num_subcores=NS),
        compiler_params=pltpu.CompilerParams(use_tc_tiling_on_sc=False),
        scratch_types=[
            pltpu.VMEM((NCHUNK, CH), jnp.int32),
            pltpu.VMEM((NBUF, CH, D), jnp.float32),
            pltpu.VMEM((NBUF, CH), jnp.float32),
            [pltpu.SemaphoreType.DMA] * NBUF,
            [pltpu.SemaphoreType.DMA] * NBUF,
        ],
    )


def _tc_body(emb_ref, fc_ref, W1_ref, b1_ref, W2_ref, b2_ref, W3_ref, b3_ref,
             Wlin_ref, blin_ref, out_ref):
    h0 = emb_ref[...]

    j = lax.broadcasted_iota(jnp.int32, (DEEP_IN, D), 0)
    d = lax.broadcasted_iota(jnp.int32, (DEEP_IN, D), 1)
    S = jnp.where(j % D == d, 1.0, 0.0).astype(jnp.float32)
    se = lax.dot_general(h0, S, (((1,), (0,)), ((), ())),
                         preferred_element_type=jnp.float32)
    ss = lax.dot_general(h0 * h0, S, (((1,), (0,)), ((), ())),
                         preferred_element_type=jnp.float32)
    fm = 0.5 * jnp.sum(se * se - ss, axis=1, keepdims=True)

    lin = jnp.sum(fc_ref[...], axis=1, keepdims=True) * Wlin_ref[0, 0] + blin_ref[0, 0]

    h1 = jnp.maximum(
        lax.dot_general(h0, W1_ref[...], (((1,), (1,)), ((), ())),
                        preferred_element_type=jnp.float32) + b1_ref[...], 0.0)
    h2 = jnp.maximum(
        lax.dot_general(h1, W2_ref[...], (((1,), (1,)), ((), ())),
                        preferred_element_type=jnp.float32) + b2_ref[...], 0.0)
    h3 = jnp.sum(h2 * W3_ref[...], axis=1, keepdims=True) + b3_ref[0, 0]

    out_ref[...] = lin + fm + h3


def _tc_forward(emb_flat, fc_mat, W1, b1, W2, b2, W3, b3, W_lin, b_lin):
    bB = 1024
    grid = (BS // bB,)
    return pl.pallas_call(
        _tc_body,
        grid=grid,
        in_specs=[
            pl.BlockSpec((bB, DEEP_IN), lambda i: (i, 0)),
            pl.BlockSpec((bB, F), lambda i: (i, 0)),
            pl.BlockSpec((H1, DEEP_IN), lambda i: (0, 0)),
            pl.BlockSpec((1, H1), lambda i: (0, 0)),
            pl.BlockSpec((H2, H1), lambda i: (0, 0)),
            pl.BlockSpec((1, H2), lambda i: (0, 0)),
            pl.BlockSpec((1, H2), lambda i: (0, 0)),
            pl.BlockSpec(memory_space=pltpu.SMEM),
            pl.BlockSpec(memory_space=pltpu.SMEM),
            pl.BlockSpec(memory_space=pltpu.SMEM),
        ],
        out_specs=pl.BlockSpec((bB, 1), lambda i: (i, 0)),
        out_shape=jax.ShapeDtypeStruct((BS, 1), jnp.float32),
    )(emb_flat, fc_mat, W1, b1, W2, b2, W3, b3, W_lin, b_lin)


def kernel(x, emb_table, fc_table, W_lin, b_lin, W1, b1, W2, b2, W3, b3):
    x_idx = x.astype(jnp.int32).reshape(N // CH, CH)
    fc_flat = fc_table.reshape(-1)
    b1r = b1.reshape(1, H1)
    b2r = b2.reshape(1, H2)
    b3r = b3.reshape(1, 1)
    blinr = b_lin.reshape(1, 1)
    sc = _make_sc_gather()
    rows_per_slice = NSL // CH
    outs = []
    for s in range(SLICES):
        xs = lax.slice_in_dim(x_idx, s * rows_per_slice, (s + 1) * rows_per_slice, axis=0)
        emb_s, fc_s = sc(xs, emb_table, fc_flat)
        outs.append(_tc_forward(
            emb_s.reshape(BS, DEEP_IN), fc_s.reshape(BS, F),
            W1, b1r, W2, b2r, W3, b3r, W_lin, blinr,
        ))
    return jnp.concatenate(outs, axis=0)
